# baseline (speedup 1.0000x reference)
import jax
import jax.numpy as jnp
from jax.experimental import pallas as pl
from jax.experimental.pallas import tpu as pltpu

K = 19


def _partial_kernel(feats_ref, labels_ref, out_ref, cnt_ref, *, bn):
    j = pl.program_id(1)

    feats = feats_ref[0]
    labels = labels_ref[0]
    classes = jax.lax.broadcasted_iota(jnp.int32, (K, bn), 0)
    onehot = (labels == classes).astype(jnp.float32)

    partial = jax.lax.dot_general(
        onehot, feats,
        dimension_numbers=(((1,), (1,)), ((), ())),
        preferred_element_type=jnp.float32,
    )
    cnt_partial = jnp.sum(onehot, axis=1).reshape(K, 1)

    @pl.when(j == 0)
    def _init():
        out_ref[0] = partial
        cnt_ref[0] = cnt_partial

    @pl.when(j > 0)
    def _acc():
        out_ref[0] += partial
        cnt_ref[0] += cnt_partial


def _finalize_kernel(sums_ref, cnt_ref, proto_ref, count_ref):
    sums = jnp.sum(sums_ref[...], axis=0)
    cnt = jnp.sum(cnt_ref[...], axis=0)
    denom = jnp.maximum(cnt, 1.0)
    proto_ref[...] = jnp.where(cnt > 0.0, sums / denom, jnp.zeros_like(sums))
    count_ref[...] = cnt


def kernel(features, labels):
    B, C, H, W = features.shape
    N = H * W
    bn = 8192
    nb = N // bn

    feats3 = features.reshape(B, C, N)
    labels3 = labels.reshape(B * nb, 1, bn)

    psums, pcnts = pl.pallas_call(
        lambda f, l, o, c: _partial_kernel(f, l, o, c, bn=bn),
        grid=(B, nb),
        in_specs=[
            pl.BlockSpec((1, C, bn), lambda b, j: (b, 0, j)),
            pl.BlockSpec((1, 1, bn), lambda b, j: (b * nb + j, 0, 0)),
        ],
        out_specs=[
            pl.BlockSpec((1, K, C), lambda b, j: (b, 0, 0)),
            pl.BlockSpec((1, K, 1), lambda b, j: (b, 0, 0)),
        ],
        out_shape=[
            jax.ShapeDtypeStruct((B, K, C), jnp.float32),
            jax.ShapeDtypeStruct((B, K, 1), jnp.float32),
        ],
        compiler_params=pltpu.CompilerParams(
            dimension_semantics=("parallel", "arbitrary"),
        ),
    )(feats3, labels3)

    protos, counts = pl.pallas_call(
        _finalize_kernel,
        out_shape=[
            jax.ShapeDtypeStruct((K, C), jnp.float32),
            jax.ShapeDtypeStruct((K, 1), jnp.float32),
        ],
    )(psums, pcnts)

    return protos, counts.reshape(K)


# speedup vs baseline: 1.0049x; 1.0049x over previous
import jax
import jax.numpy as jnp
from jax.experimental import pallas as pl
from jax.experimental.pallas import tpu as pltpu

K = 19


def _partial_kernel(feats_ref, labels_ref, out_ref, cnt_ref, *, bn):
    j = pl.program_id(1)

    feats = feats_ref[0]
    labels = labels_ref[0]
    classes = jax.lax.broadcasted_iota(jnp.int32, (K, bn), 0)
    onehot = (labels == classes).astype(jnp.float32)

    partial = jax.lax.dot_general(
        onehot, feats,
        dimension_numbers=(((1,), (1,)), ((), ())),
        preferred_element_type=jnp.float32,
    )
    cnt_partial = jnp.sum(onehot, axis=1).reshape(K, 1)

    @pl.when(j == 0)
    def _init():
        out_ref[0] = partial
        cnt_ref[0] = cnt_partial

    @pl.when(j > 0)
    def _acc():
        out_ref[0] += partial
        cnt_ref[0] += cnt_partial


def _finalize_kernel(sums_ref, cnt_ref, proto_ref, count_ref):
    sums = jnp.sum(sums_ref[...], axis=0)
    cnt = jnp.sum(cnt_ref[...], axis=0)
    denom = jnp.maximum(cnt, 1.0)
    proto_ref[...] = jnp.where(cnt > 0.0, sums / denom, jnp.zeros_like(sums))
    count_ref[...] = cnt


def kernel(features, labels):
    B, C, H, W = features.shape
    N = H * W
    bn = 16384
    nb = N // bn

    feats3 = features.reshape(B, C, N)
    labels3 = labels.reshape(B * nb, 1, bn)

    psums, pcnts = pl.pallas_call(
        lambda f, l, o, c: _partial_kernel(f, l, o, c, bn=bn),
        grid=(B, nb),
        in_specs=[
            pl.BlockSpec((1, C, bn), lambda b, j: (b, 0, j)),
            pl.BlockSpec((1, 1, bn), lambda b, j: (b * nb + j, 0, 0)),
        ],
        out_specs=[
            pl.BlockSpec((1, K, C), lambda b, j: (b, 0, 0)),
            pl.BlockSpec((1, K, 1), lambda b, j: (b, 0, 0)),
        ],
        out_shape=[
            jax.ShapeDtypeStruct((B, K, C), jnp.float32),
            jax.ShapeDtypeStruct((B, K, 1), jnp.float32),
        ],
        compiler_params=pltpu.CompilerParams(
            dimension_semantics=("parallel", "arbitrary"),
        ),
    )(feats3, labels3)

    protos, counts = pl.pallas_call(
        _finalize_kernel,
        out_shape=[
            jax.ShapeDtypeStruct((K, C), jnp.float32),
            jax.ShapeDtypeStruct((K, 1), jnp.float32),
        ],
    )(psums, pcnts)

    return protos, counts.reshape(K)


# speedup vs baseline: 1.0089x; 1.0040x over previous
import jax
import jax.numpy as jnp
from jax.experimental import pallas as pl
from jax.experimental.pallas import tpu as pltpu

K = 19


def _partial_kernel(f0_ref, f1_ref, labels_ref, out_ref, cnt_ref, *, bn):
    j = pl.program_id(1)

    labels = labels_ref[0]
    classes = jax.lax.broadcasted_iota(jnp.int32, (K, bn), 0)
    onehot = (labels == classes).astype(jnp.float32)

    def _dot(feats):
        return jax.lax.dot_general(
            onehot, feats,
            dimension_numbers=(((1,), (1,)), ((), ())),
            preferred_element_type=jnp.float32,
        )
    p0 = _dot(f0_ref[0])
    p1 = _dot(f1_ref[0])
    partial = jnp.concatenate([p0, p1], axis=1)
    cnt_partial = jnp.sum(onehot, axis=1).reshape(K, 1)

    @pl.when(j == 0)
    def _init():
        out_ref[0] = partial
        cnt_ref[0] = cnt_partial

    @pl.when(j > 0)
    def _acc():
        out_ref[0] += partial
        cnt_ref[0] += cnt_partial


def _finalize_kernel(sums_ref, cnt_ref, proto_ref, count_ref):
    sums = jnp.sum(sums_ref[...], axis=0)
    cnt = jnp.sum(cnt_ref[...], axis=0)
    denom = jnp.maximum(cnt, 1.0)
    proto_ref[...] = jnp.where(cnt > 0.0, sums / denom, jnp.zeros_like(sums))
    count_ref[...] = cnt


def kernel(features, labels):
    B, C, H, W = features.shape
    N = H * W
    bn = 8192
    nb = N // bn

    feats3 = features.reshape(B, C, N)
    labels3 = labels.reshape(B * nb, 1, bn)

    psums, pcnts = pl.pallas_call(
        lambda f0, f1, l, o, c: _partial_kernel(f0, f1, l, o, c, bn=bn),
        grid=(B, nb),
        in_specs=[
            pl.BlockSpec((1, C // 2, bn), lambda b, j: (b, 0, j)),
            pl.BlockSpec((1, C // 2, bn), lambda b, j: (b, 1, j)),
            pl.BlockSpec((1, 1, bn), lambda b, j: (b * nb + j, 0, 0)),
        ],
        out_specs=[
            pl.BlockSpec((1, K, C), lambda b, j: (b, 0, 0)),
            pl.BlockSpec((1, K, 1), lambda b, j: (b, 0, 0)),
        ],
        out_shape=[
            jax.ShapeDtypeStruct((B, K, C), jnp.float32),
            jax.ShapeDtypeStruct((B, K, 1), jnp.float32),
        ],
        compiler_params=pltpu.CompilerParams(
            dimension_semantics=("parallel", "arbitrary"),
        ),
    )(feats3, feats3, labels3)

    protos, counts = pl.pallas_call(
        _finalize_kernel,
        out_shape=[
            jax.ShapeDtypeStruct((K, C), jnp.float32),
            jax.ShapeDtypeStruct((K, 1), jnp.float32),
        ],
    )(psums, pcnts)

    return protos, counts.reshape(K)
